# baseline (speedup 1.0000x reference)
import functools

import jax
import jax.numpy as jnp
from jax import lax
from jax.experimental import pallas as pl
from jax.experimental.pallas import tpu as pltpu
from jax.experimental.pallas import tpu_sc as plsc

EMBED = 64
FACTOR = 8.0

NUM_CORES = 2
NUM_SUBCORES = 16
NUM_WORKERS = NUM_CORES * NUM_SUBCORES
LANES = 16

VB = 128
COMPILER = pltpu.CompilerParams(
    use_tc_tiling_on_sc=True, needs_layout_passes=False
)


def _iota16():
    return lax.iota(jnp.int32, LANES)


@functools.partial(jax.jit, static_argnums=(2,))
def _pack_table(tableT, tailT, n_full):
    vocab = tableT.shape[1]
    n_tail = vocab - n_full * VB
    assert n_tail == VB // 2
    per_w = n_full // NUM_WORKERS
    n_extra = n_full - per_w * NUM_WORKERS
    mesh = plsc.VectorSubcoreMesh(core_axis_name="c", subcore_axis_name="s")

    @functools.partial(
        pl.kernel,
        out_type=jax.ShapeDtypeStruct((n_full * (VB // 2) + VB // 2, VB),
                                      jnp.float32),
        mesh=mesh,
        scratch_types=[
            pltpu.VMEM((EMBED, VB), jnp.float32),
            pltpu.VMEM((EMBED, VB), jnp.float32),
            pltpu.VMEM((EMBED, VB), jnp.float32),
            pltpu.VMEM((EMBED, VB), jnp.float32),
            pltpu.SemaphoreType.DMA,
            pltpu.SemaphoreType.DMA,
            pltpu.SemaphoreType.DMA,
            pltpu.SemaphoreType.DMA,
        ],
        compiler_params=COMPILER,
    )
    def body(tableT_hbm, tailT_hbm, tl_hbm, s0, s1, w0, w1,
             gs0, gs1, ws0, ws1):
        wid = lax.axis_index("s") * NUM_CORES + lax.axis_index("c")
        iota = _iota16()
        rows = [iota + 16 * (q % 4) for q in range(8)]

        def stage(b, sbuf, gsem):
            pltpu.async_copy(tableT_hbm.at[:, pl.ds(b * VB, VB)], sbuf, gsem)

        def drain_stage(sbuf, gsem):
            pltpu.make_async_copy(
                tableT_hbm.at[:, pl.ds(0, VB)], sbuf, gsem
            ).wait()

        def transpose_block(sbuf, wbuf, width):
            def step_p(p, c2):
                for q in range(8):
                    col = jnp.full((LANES,), 2 * p + q // 4, jnp.int32)
                    v = plsc.load_gather(sbuf, (rows[q], col))
                    wbuf[p, pl.ds(16 * q, 16)] = v * FACTOR
                return c2

            lax.fori_loop(0, width // 2, step_p, 0, unroll=8)

        def fire_write(b, wbuf, wsem, nrows=EMBED):
            pltpu.async_copy(
                wbuf.at[pl.ds(0, nrows)],
                tl_hbm.at[pl.ds(b * (VB // 2), nrows)],
                wsem,
            )

        def drain_write(wbuf, wsem, nrows=EMBED):
            pltpu.make_async_copy(
                wbuf.at[pl.ds(0, nrows)],
                tl_hbm.at[pl.ds(0, nrows)],
                wsem,
            ).wait()

        stage(wid, s0, gs0)

        def step(j, carry):
            i0 = 2 * j
            b0 = wid + NUM_WORKERS * i0
            b1 = b0 + NUM_WORKERS

            drain_stage(s0, gs0)

            @pl.when(i0 + 1 < per_w)
            def _():
                stage(b1, s1, gs1)

            @pl.when(j > 0)
            def _():
                drain_write(w0, ws0)

            transpose_block(s0, w0, VB)
            fire_write(b0, w0, ws0)

            @pl.when(i0 + 1 < per_w)
            def _():
                drain_stage(s1, gs1)

                @pl.when(i0 + 2 < per_w)
                def _():
                    stage(b1 + NUM_WORKERS, s0, gs0)

                @pl.when(j > 0)
                def _():
                    drain_write(w1, ws1)

                transpose_block(s1, w1, VB)
                fire_write(b1, w1, ws1)

            return carry

        lax.fori_loop(0, (per_w + 1) // 2, step, 0)
        drain_write(w0, ws0)

        @pl.when(per_w > 1)
        def _():
            drain_write(w1, ws1)

        @pl.when(wid < n_extra)
        def _():
            b = per_w * NUM_WORKERS + wid
            stage(b, s0, gs0)
            drain_stage(s0, gs0)
            transpose_block(s0, w0, VB)
            fire_write(b, w0, ws0)
            drain_write(w0, ws0)

        @pl.when(wid == n_extra)
        def _():
            pltpu.async_copy(tailT_hbm, s0, gs0)
            drain_stage(s0, gs0)
            transpose_block(s0, w0, n_tail)
            fire_write(n_full, w0, ws0, nrows=n_tail // 2)
            drain_write(w0, ws0, nrows=n_tail // 2)

    return body(tableT, tailT)


@functools.partial(jax.jit, static_argnums=(2,))
def _gather_out(srcT, tableL, cols_per_worker):
    row_len, n_rows = srcT.shape
    assert row_len % 2 == 0
    mesh = plsc.VectorSubcoreMesh(core_axis_name="c", subcore_axis_name="s")

    @functools.partial(
        pl.kernel,
        out_type=jax.ShapeDtypeStruct((row_len, EMBED, n_rows), jnp.float32),
        mesh=mesh,
        scratch_types=[
            pltpu.VMEM((row_len, VB), jnp.int32),
            pltpu.VMEM((VB, VB), jnp.float32),
            pltpu.VMEM((VB, VB), jnp.float32),
            pltpu.VMEM((EMBED, VB), jnp.float32),
            pltpu.VMEM((EMBED, VB), jnp.float32),
            pltpu.VMEM((VB,), jnp.int32),
            pltpu.VMEM((VB,), jnp.int32),
            pltpu.SemaphoreType.DMA,
            pltpu.SemaphoreType.DMA,
            pltpu.SemaphoreType.DMA,
            pltpu.SemaphoreType.DMA,
            pltpu.SemaphoreType.DMA,
        ],
        compiler_params=COMPILER,
    )
    def body(tableL_hbm, srcT_hbm, outT_hbm, idxT, g0, g1, w0, w1,
             h0, h1, isem, gs0, gs1, ws0, ws1):
        wid = lax.axis_index("s") * NUM_CORES + lax.axis_index("c")
        col0 = wid * cols_per_worker
        iota = _iota16()
        rows = [iota + 16 * t for t in range(8)]

        pltpu.async_copy(srcT_hbm.at[:, pl.ds(col0, VB)], idxT, isem)
        pltpu.make_async_copy(
            srcT_hbm.at[:, pl.ds(0, VB)], idxT, isem
        ).wait()

        def fire_gather(c, hbuf, gbuf, gsem):
            def half(t, c2):
                hbuf[pl.ds(16 * t, 16)] = lax.shift_right_logical(
                    idxT[c, pl.ds(16 * t, 16)], 1
                )
                return c2

            lax.fori_loop(0, 8, half, 0, unroll=8)
            pltpu.async_copy(tableL_hbm.at[hbuf], gbuf, gsem)

        def drain_gather(gbuf, gsem):
            pltpu.make_async_copy(
                tableL_hbm.at[pl.ds(0, VB)], gbuf, gsem
            ).wait()

        def build(c, gbuf, wbuf):
            def step_t(t, c2):
                idxv = idxT[c, pl.ds(16 * t, 16)]
                colbase = lax.mul(lax.bitwise_and(idxv, 1), EMBED)
                rowv = iota + 16 * t
                for e in range(EMBED):
                    v = plsc.load_gather(gbuf, (rowv, colbase + e))
                    wbuf[e, pl.ds(16 * t, 16)] = v
                return c2

            lax.fori_loop(0, 8, step_t, 0)

        def fire_write(c, wbuf, wsem):
            pltpu.async_copy(
                wbuf, outT_hbm.at[c, :, pl.ds(col0, VB)], wsem
            )

        def drain_write(wbuf, wsem):
            pltpu.make_async_copy(
                wbuf, outT_hbm.at[0, :, pl.ds(0, VB)], wsem
            ).wait()

        fire_gather(0, h0, g0, gs0)

        def step(j, carry):
            c0 = 2 * j
            c1 = c0 + 1

            drain_gather(g0, gs0)
            fire_gather(c1, h1, g1, gs1)

            @pl.when(j > 0)
            def _():
                drain_write(w0, ws0)

            build(c0, g0, w0)
            fire_write(c0, w0, ws0)

            drain_gather(g1, gs1)

            @pl.when(c1 + 1 < row_len)
            def _():
                fire_gather(c1 + 1, h0, g0, gs0)

            @pl.when(j > 0)
            def _():
                drain_write(w1, ws1)

            build(c1, g1, w1)
            fire_write(c1, w1, ws1)
            return carry

        lax.fori_loop(0, row_len // 2, step, 0)
        drain_write(w0, ws0)
        drain_write(w1, ws1)

    return body(tableL, srcT)


def kernel(src, table):
    n_rows, row_len = src.shape
    vocab, embed = table.shape
    assert embed == EMBED and n_rows % (NUM_WORKERS * VB) == 0
    n_full = vocab // VB
    tableT = table.T
    tailT = jnp.pad(tableT[:, n_full * VB:],
                    ((0, 0), (0, VB - (vocab - n_full * VB))))
    tableL = _pack_table(tableT, tailT, n_full)
    outT = _gather_out(src.T, tableL, n_rows // NUM_WORKERS)
    return jnp.transpose(outT, (2, 0, 1))


# speedup vs baseline: 2.1710x; 2.1710x over previous
import functools

import jax
import jax.numpy as jnp
from jax import lax
from jax.experimental import pallas as pl
from jax.experimental.pallas import tpu as pltpu
from jax.experimental.pallas import tpu_sc as plsc

EMBED = 64
FACTOR = 8.0

NUM_CORES = 2
NUM_SUBCORES = 16
NUM_WORKERS = NUM_CORES * NUM_SUBCORES
LANES = 16
VB = 128

TILED = pltpu.CompilerParams(
    use_tc_tiling_on_sc=True, needs_layout_passes=False
)
LINEAR = pltpu.CompilerParams(
    use_tc_tiling_on_sc=False, needs_layout_passes=False
)

PACK_ROWS = 256


@jax.jit
def _pack_table(table):
    vocab = table.shape[0]
    n_chunks = vocab // PACK_ROWS
    tail = vocab - n_chunks * PACK_ROWS
    chunks = (n_chunks // NUM_WORKERS) & ~1
    n_extra = n_chunks - chunks * NUM_WORKERS
    assert tail % 16 == 0
    mesh = plsc.VectorSubcoreMesh(core_axis_name="c", subcore_axis_name="s")

    @functools.partial(
        pl.kernel,
        out_type=jax.ShapeDtypeStruct((vocab // 2, VB), jnp.float32),
        mesh=mesh,
        scratch_types=[
            pltpu.VMEM((PACK_ROWS, EMBED), jnp.float32),
            pltpu.VMEM((PACK_ROWS, EMBED), jnp.float32),
            pltpu.VMEM((PACK_ROWS // 2, VB), jnp.float32),
            pltpu.VMEM((PACK_ROWS // 2, VB), jnp.float32),
            pltpu.SemaphoreType.DMA,
            pltpu.SemaphoreType.DMA,
            pltpu.SemaphoreType.DMA,
            pltpu.SemaphoreType.DMA,
        ],
        compiler_params=TILED,
    )
    def body(table_hbm, tl_hbm, s0, s1, w0, w1, gs0, gs1, ws0, ws1):
        wid = lax.axis_index("s") * NUM_CORES + lax.axis_index("c")
        row0 = wid * chunks * PACK_ROWS

        def stage(i, sbuf, gsem, nrows=PACK_ROWS):
            pltpu.async_copy(
                table_hbm.at[pl.ds(row0 + i * PACK_ROWS, nrows)],
                sbuf.at[pl.ds(0, nrows)], gsem,
            )

        def drain_stage(sbuf, gsem, nrows=PACK_ROWS):
            pltpu.make_async_copy(
                table_hbm.at[pl.ds(0, nrows)], sbuf.at[pl.ds(0, nrows)], gsem
            ).wait()

        def pack(sbuf, wbuf, nrows=PACK_ROWS):
            def step_r(r, c2):
                half = lax.mul(lax.rem(r, 2), EMBED)
                for k in range(EMBED // LANES):
                    v = sbuf[r, pl.ds(16 * k, 16)]
                    wbuf[lax.div(r, 2), pl.ds(half + 16 * k, 16)] = v * FACTOR
                return c2

            lax.fori_loop(0, nrows, step_r, 0, unroll=8)

        out_row0 = wid * chunks * (PACK_ROWS // 2)

        def fire_write(i, wbuf, wsem, nrows=PACK_ROWS):
            pltpu.async_copy(
                wbuf.at[pl.ds(0, nrows // 2)],
                tl_hbm.at[pl.ds(out_row0 + i * (PACK_ROWS // 2), nrows // 2)],
                wsem,
            )

        def drain_write(wbuf, wsem, nrows=PACK_ROWS):
            pltpu.make_async_copy(
                wbuf.at[pl.ds(0, nrows // 2)],
                tl_hbm.at[pl.ds(0, nrows // 2)], wsem
            ).wait()

        stage(0, s0, gs0)

        def step(j, carry):
            i0 = 2 * j
            i1 = i0 + 1

            drain_stage(s0, gs0)
            stage(i1, s1, gs1)

            @pl.when(j > 0)
            def _():
                drain_write(w0, ws0)

            pack(s0, w0)
            fire_write(i0, w0, ws0)

            drain_stage(s1, gs1)

            @pl.when(i1 + 1 < chunks)
            def _():
                stage(i1 + 1, s0, gs0)

            @pl.when(j > 0)
            def _():
                drain_write(w1, ws1)

            pack(s1, w1)
            fire_write(i1, w1, ws1)
            return carry

        lax.fori_loop(0, chunks // 2, step, 0)
        drain_write(w0, ws0)
        drain_write(w1, ws1)

        @pl.when(wid < n_extra)
        def _():
            base = chunks * NUM_WORKERS * PACK_ROWS
            i = (base - row0) // PACK_ROWS + wid
            stage(i, s0, gs0)
            drain_stage(s0, gs0)
            pack(s0, w0)
            fire_write(i, w0, ws0)
            drain_write(w0, ws0)

        @pl.when((wid == n_extra) & (tail > 0))
        def _():
            base = n_chunks * PACK_ROWS
            i = (base - row0) // PACK_ROWS
            stage(i, s0, gs0, nrows=tail)
            drain_stage(s0, gs0, nrows=tail)
            pack(s0, w0, nrows=tail)
            fire_write(i, w0, ws0, nrows=tail)
            drain_write(w0, ws0, nrows=tail)

    return body(table)


@jax.jit
def _gather_out(srcT, tableL):
    row_len, n_rows = srcT.shape
    assert n_rows == NUM_WORKERS * VB and row_len % 2 == 0
    mesh = plsc.VectorSubcoreMesh(core_axis_name="c", subcore_axis_name="s")

    @functools.partial(
        pl.kernel,
        out_type=jax.ShapeDtypeStruct(
            (row_len, EMBED // 8, n_rows // VB, 8, VB), jnp.float32),
        mesh=mesh,
        scratch_types=[
            pltpu.VMEM((row_len, VB), jnp.int32),
            pltpu.VMEM((VB, VB), jnp.float32),
            pltpu.VMEM((VB, VB), jnp.float32),
            pltpu.VMEM((EMBED // 8, 8, VB + 1), jnp.float32),
            pltpu.VMEM((EMBED // 8, 8, VB + 1), jnp.float32),
            pltpu.VMEM((VB,), jnp.int32),
            pltpu.VMEM((VB,), jnp.int32),
            pltpu.SemaphoreType.DMA,
            pltpu.SemaphoreType.DMA,
            pltpu.SemaphoreType.DMA,
            pltpu.SemaphoreType.DMA,
            pltpu.SemaphoreType.DMA,
        ],
        compiler_params=LINEAR,
    )
    def body(tableL_hbm, srcT_hbm, out4_hbm, idxT, g0, g1, w0, w1,
             h0, h1, isem, gs0, gs1, ws0, ws1):
        wid = lax.axis_index("s") * NUM_CORES + lax.axis_index("c")
        col0 = wid * VB
        iota = lax.iota(jnp.int32, LANES)

        pltpu.async_copy(srcT_hbm.at[:, pl.ds(col0, VB)], idxT, isem)
        pltpu.make_async_copy(
            srcT_hbm.at[:, pl.ds(0, VB)], idxT, isem
        ).wait()

        r1 = [lax.shift_right_logical(iota + 16 * k, 3) for k in range(4)]
        r2 = [lax.bitwise_and(iota + 16 * k, 7) for k in range(4)]

        def fire_gather(c, hbuf, gbuf, gsem):
            def half(t, c2):
                hbuf[pl.ds(16 * t, 16)] = lax.shift_right_logical(
                    idxT[c, pl.ds(16 * t, 16)], 1
                )
                return c2

            lax.fori_loop(0, 8, half, 0, unroll=8)
            pltpu.async_copy(tableL_hbm.at[hbuf], gbuf, gsem)

        def drain_gather(gbuf, gsem):
            pltpu.make_async_copy(
                tableL_hbm.at[pl.ds(0, VB)], gbuf, gsem
            ).wait()

        def build(c, gbuf, wbuf):
            def step_t(t, c2):
                parv = lax.mul(
                    lax.bitwise_and(idxT[c, pl.ds(16 * t, 16)], 1), EMBED
                )
                for j in range(LANES):
                    b = 16 * t + j
                    off = parv[j]
                    colv = jnp.full((LANES,), b, jnp.int32)
                    for k in range(EMBED // LANES):
                        v = gbuf[b, pl.ds(off + 16 * k, 16)]
                        plsc.store_scatter(wbuf, (r1[k], r2[k], colv), v)
                return c2

            lax.fori_loop(0, 8, step_t, 0)

        def fire_write(c, wbuf, wsem):
            pltpu.async_copy(
                wbuf.at[:, :, pl.ds(0, VB)], out4_hbm.at[c, :, wid], wsem
            )

        def drain_write(wbuf, wsem):
            pltpu.make_async_copy(
                wbuf.at[:, :, pl.ds(0, VB)], out4_hbm.at[0, :, 0], wsem
            ).wait()

        fire_gather(0, h0, g0, gs0)

        def step(j, carry):
            c0 = 2 * j
            c1 = c0 + 1

            drain_gather(g0, gs0)
            fire_gather(c1, h1, g1, gs1)

            @pl.when(j > 0)
            def _():
                drain_write(w0, ws0)

            build(c0, g0, w0)
            fire_write(c0, w0, ws0)

            drain_gather(g1, gs1)

            @pl.when(c1 + 1 < row_len)
            def _():
                fire_gather(c1 + 1, h0, g0, gs0)

            @pl.when(j > 0)
            def _():
                drain_write(w1, ws1)

            build(c1, g1, w1)
            fire_write(c1, w1, ws1)
            return carry

        lax.fori_loop(0, row_len // 2, step, 0)
        drain_write(w0, ws0)
        drain_write(w1, ws1)

    return body(tableL, srcT)


def kernel(src, table):
    n_rows, row_len = src.shape
    vocab, embed = table.shape
    assert embed == EMBED and n_rows == NUM_WORKERS * VB
    tableL = _pack_table(table)
    out4 = _gather_out(src.T, tableL)
    return jnp.reshape(
        jnp.transpose(out4, (2, 4, 0, 1, 3)), (n_rows, row_len, embed)
    )
